# baseline (speedup 1.0000x reference)
import functools

import jax
import jax.numpy as jnp
import numpy as np
from jax import lax
from jax.experimental import pallas as pl
from jax.experimental.pallas import tpu as pltpu
from jax.experimental.pallas import tpu_sc as plsc

_NC = 2
_NS = 16
_NW = _NC * _NS
_QTR = 100
_NBUF = 8
_CH = 66
_LANES = 16
_D = 128
_VPR = _D // _LANES

_PERM = np.array([32 * (c // 32) + 2 * (c % 16) + ((c % 32) // 16)
                  for c in range(_D)], dtype=np.int32)


def _sc_pool_call(s_pad, table_rows):
    seg_w = s_pad // _NW
    nchunk = seg_w // _CH
    ntasks = 4 * _CH
    mesh = plsc.VectorSubcoreMesh(core_axis_name="c", subcore_axis_name="s")

    def body(ids_hbm, table_hbm, out_hbm, idx_v, rows_v, stage_v, sems):
        wid = lax.axis_index("s") * _NC + lax.axis_index("c")
        seg0 = wid * seg_w

        def chunk_body(cidx, carry):
            cbase = seg0 + cidx * _CH
            pltpu.sync_copy(ids_hbm.at[pl.ds(cbase, _CH)], idx_v)
            for b in range(_NBUF):
                pltpu.async_copy(
                    table_hbm.at[idx_v.at[b // 4, b % 4]],
                    rows_v.at[b], sems.at[b])

            def group_body(g, carry2):
                t0 = g * _NBUF
                s0 = g * (_NBUF // 4)
                accs = None
                for b in range(_NBUF):
                    s = s0 + b // 4
                    h = b % 4
                    pltpu.make_async_copy(
                        table_hbm.at[idx_v.at[s, h]],
                        rows_v.at[b], sems.at[b]).wait()
                    if h == 0:
                        accs = tuple(jnp.zeros((_LANES,), jnp.float32)
                                     for _ in range(2 * _VPR))

                    def acc_body(j, a, _b=b):
                        a = list(a)
                        for r in range(4):
                            row = j * 4 + r
                            bank = (r % 2) * _VPR
                            for g2 in range(_VPR // 2):
                                w = rows_v[_b, row,
                                           pl.ds(g2 * _LANES, _LANES)]
                                lo = lax.bitcast_convert_type(
                                    lax.shift_left(w, 16), jnp.float32)
                                hi = lax.bitcast_convert_type(
                                    lax.bitwise_and(w, jnp.int32(-65536)),
                                    jnp.float32)
                                a[bank + 2 * g2] = a[bank + 2 * g2] + lo
                                a[bank + 2 * g2 + 1] = (
                                    a[bank + 2 * g2 + 1] + hi)
                        return tuple(a)

                    accs = lax.fori_loop(0, _QTR // 4, acc_body, accs)
                    if h == 3:
                        for k in range(2 * _VPR):
                            stage_v[s, pl.ds(k * _LANES, _LANES)] = accs[k]
                    nxt_t = t0 + b + _NBUF
                    nxt_s = s + _NBUF // 4
                    @pl.when(nxt_t < ntasks)
                    def _issue(_b=b, _h=h, _s=nxt_s):
                        pltpu.async_copy(
                            table_hbm.at[idx_v.at[_s, _h]],
                            rows_v.at[_b], sems.at[_b])
                return carry2

            lax.fori_loop(0, ntasks // _NBUF, group_body, 0)
            pltpu.sync_copy(stage_v, out_hbm.at[pl.ds(cbase, _CH)])
            return carry

        lax.fori_loop(0, nchunk, chunk_body, 0)

    return pl.kernel(
        body,
        out_type=jax.ShapeDtypeStruct((s_pad, 2 * _D), jnp.float32),
        mesh=mesh,
        compiler_params=pltpu.CompilerParams(use_tc_tiling_on_sc=False),
        scratch_types=[
            pltpu.VMEM((_CH, 4, _QTR), jnp.int32),
            pltpu.VMEM((_NBUF, _QTR, _D // 2), jnp.int32),
            pltpu.VMEM((_CH, 2 * _D), jnp.float32),
            pltpu.SemaphoreType.DMA((_NBUF,)),
        ],
    )


_PREC = lax.Precision.HIGHEST


def _dense_call(bsz, blk, nb, few, n_blk_off):
    grid = (bsz // blk,)
    d = _D
    dm = 2 * d
    dh = 2 * dm
    fnb = float(nb)

    def body(qlp, qrp, qld, qrd, slp, srp, sld, srd,
             gwT, gb, w1T, b1, w2T, b2, lng, lnb, wihT, whhT, bih, bhh,
             out_ref):
        dot = functools.partial(jnp.dot, preferred_element_type=jnp.float32,
                                precision=_PREC)
        gwTv = gwT[...]
        gbv = gb[...]

        def nenc(p, deg):
            return jnp.tanh((dot(p, gwTv) + fnb * gbv) / deg)

        lngv = lng[...]
        lnbv = lnb[...]
        w1Tv = w1T[...]
        b1v = b1[...]
        w2Tv = w2T[...]
        b2v = b2[...]

        def senc(x):
            hh = jnp.maximum(dot(x, w1Tv) + b1v, 0.0)
            hh = dot(hh, w2Tv) + b2v
            z = hh + x
            mu = jnp.mean(z, axis=-1, keepdims=True)
            var = jnp.sum((z - mu) ** 2, axis=-1, keepdims=True) / (dm - 1)
            return (z - mu) / (jnp.sqrt(var) + 1e-3) * lngv + lnbv

        sl = nenc(slp[...], sld[...])
        sr = nenc(srp[...], srd[...])
        sgall = senc(jnp.concatenate([sl, sr], axis=1))
        smask = (lax.broadcasted_iota(jnp.int32, sgall.shape, 0)
                 < few).astype(jnp.float32)
        sg = jnp.sum(sgall * smask, axis=0, keepdims=True) / float(few)

        ql = nenc(qlp[...], qld[...])
        qr = nenc(qrp[...], qrd[...])
        qg = senc(jnp.concatenate([ql, qr], axis=1))

        whhTv = whhT[...]
        xp = dot(qg, wihT[...]) + bih[...] + bhh[...]
        sterm = dot(sg, whhTv[dm:dh])
        whhTl = whhTv[0:dm]
        h = None
        c = None
        for step in range(4):
            gates = xp if step == 0 else xp + dot(h, whhTl) + sterm
            ig = jax.nn.sigmoid(gates[:, 0:dh])
            fg = jax.nn.sigmoid(gates[:, dh:2 * dh])
            gg = jnp.tanh(gates[:, 2 * dh:3 * dh])
            og = jax.nn.sigmoid(gates[:, 3 * dh:4 * dh])
            c = ig * gg if step == 0 else fg * c + ig * gg
            h = qg + (og * jnp.tanh(c))[:, 0:dm]
        out_ref[...] = jnp.sum(h * sg, axis=1, keepdims=True)

    row_spec = lambda cols: pl.BlockSpec((blk, cols), lambda i: (i, 0))
    full = lambda shape: pl.BlockSpec(shape, lambda i: (0,) * len(shape))
    return pl.pallas_call(
        body,
        grid=grid,
        in_specs=[
            pl.BlockSpec((blk, dm), lambda i: (i, 0)),
            pl.BlockSpec((blk, dm), lambda i: (i + n_blk_off, 0)),
            row_spec(1), row_spec(1),
            full((8, dm)), full((8, dm)), full((8, 1)), full((8, 1)),
            full((dm, d)), full((1, d)),
            full((dm, dh)), full((1, dh)),
            full((dh, dm)), full((1, dm)),
            full((1, dm)), full((1, dm)),
            full((dm, 4 * dh)), full((dh, 4 * dh)),
            full((1, 4 * dh)), full((1, 4 * dh)),
        ],
        out_specs=row_spec(1),
        out_shape=jax.ShapeDtypeStruct((bsz, 1), jnp.float32),
    )


def kernel(query, support, query_left_connections, query_left_degrees,
           query_right_connections, query_right_degrees,
           support_left_connections, support_left_degrees,
           support_right_connections, support_right_degrees,
           symbol_emb, gcn_w_W, gcn_w_b, se_w1, se_b1, se_w2, se_b2,
           ln_g, ln_b, qe_wih, qe_whh, qe_bih, qe_bhh):
    bsz = query_left_connections.shape[0]
    few = support_left_connections.shape[0]
    nb = query_left_connections.shape[1]

    rt = symbol_emb.shape[0]
    tb = symbol_emb.astype(jnp.bfloat16).reshape(rt, _D // 2, 2)
    table_p = lax.bitcast_convert_type(tb, jnp.int32)

    ids = jnp.concatenate([
        query_left_connections.reshape(bsz, 2 * nb),
        query_right_connections.reshape(bsz, 2 * nb),
        support_left_connections.reshape(few, 2 * nb),
        support_right_connections.reshape(few, 2 * nb),
    ], axis=0).astype(jnp.int32)
    s_raw = ids.shape[0]
    unit = _NW * _CH
    s_pad = -(-s_raw // unit) * unit
    dummy = jnp.zeros((s_pad - s_raw, 2 * nb), jnp.int32)
    dummy = (lax.broadcasted_iota(jnp.int32, dummy.shape, 0) * (2 * nb)
             + lax.broadcasted_iota(jnp.int32, dummy.shape, 1)) % rt
    ids = jnp.concatenate([ids, dummy], axis=0).reshape(s_pad, 4, _QTR)

    pooled = _sc_pool_call(s_pad, rt)(ids, table_p)

    off = 2 * bsz
    pad_s = jnp.zeros((8 - few, 2 * _D), jnp.float32)
    slp = jnp.concatenate([pooled[off:off + few], pad_s], 0)
    srp = jnp.concatenate([pooled[off + few:off + 2 * few], pad_s], 0)
    pad_d = jnp.ones((8 - few, 1), jnp.float32)
    sld = jnp.concatenate([support_left_degrees.reshape(few, 1), pad_d], 0)
    srd = jnp.concatenate([support_right_degrees.reshape(few, 1), pad_d], 0)

    gwT = gcn_w_W.T
    gw_perm = jnp.concatenate([gwT[_PERM], gwT[_D + _PERM]], axis=0)

    blk = 512
    scores = _dense_call(bsz, blk, nb, few, bsz // blk)(
        pooled, pooled,
        query_left_degrees.reshape(bsz, 1),
        query_right_degrees.reshape(bsz, 1),
        slp, srp, sld, srd,
        gw_perm, gcn_w_b.reshape(1, _D),
        se_w1.T, se_b1.reshape(1, -1),
        se_w2.T, se_b2.reshape(1, -1),
        ln_g.reshape(1, -1), ln_b.reshape(1, -1),
        qe_wih.T, qe_whh.T,
        qe_bih.reshape(1, -1), qe_bhh.reshape(1, -1),
    )
    return scores.reshape(bsz)


# speedup vs baseline: 1.2760x; 1.2760x over previous
import functools

import jax
import jax.numpy as jnp
import numpy as np
from jax import lax
from jax.experimental import pallas as pl
from jax.experimental.pallas import tpu as pltpu
from jax.experimental.pallas import tpu_sc as plsc

_NC = 2
_NS = 16
_NW = _NC * _NS
_QTR = 100
_NBUF = 8
_CH = 66
_LANES = 16
_D = 128
_VPR = _D // _LANES

_PERM = np.array([32 * (c // 32) + 2 * (c % 16) + ((c % 32) // 16)
                  for c in range(_D)], dtype=np.int32)


def _sc_pool_call(s_pad, table_rows):
    seg_w = s_pad // _NW
    nchunk = seg_w // _CH
    ntasks = 4 * _CH
    mesh = plsc.VectorSubcoreMesh(core_axis_name="c", subcore_axis_name="s")

    def body(ids_hbm, table_hbm, out_hbm, idx_v, rows_v, stage_v, sems):
        wid = lax.axis_index("s") * _NC + lax.axis_index("c")
        seg0 = wid * seg_w

        def chunk_body(cidx, carry):
            cbase = seg0 + cidx * _CH
            pltpu.sync_copy(ids_hbm.at[pl.ds(cbase, _CH)], idx_v)
            for b in range(_NBUF):
                pltpu.async_copy(
                    table_hbm.at[idx_v.at[b // 4, b % 4]],
                    rows_v.at[b], sems.at[b])

            def group_body(g, carry2):
                t0 = g * _NBUF
                s0 = g * (_NBUF // 4)
                accs = None
                for b in range(_NBUF):
                    s = s0 + b // 4
                    h = b % 4
                    pltpu.make_async_copy(
                        table_hbm.at[idx_v.at[s, h]],
                        rows_v.at[b], sems.at[b]).wait()
                    if h == 0:
                        accs = tuple(jnp.zeros((_LANES,), jnp.float32)
                                     for _ in range(2 * _VPR))

                    def acc_body(j, a, _b=b):
                        a = list(a)
                        for r in range(4):
                            row = j * 4 + r
                            bank = (r % 2) * _VPR
                            for g2 in range(_VPR // 2):
                                w = rows_v[_b, row,
                                           pl.ds(g2 * _LANES, _LANES)]
                                lo = lax.bitcast_convert_type(
                                    lax.shift_left(w, 16), jnp.float32)
                                hi = lax.bitcast_convert_type(
                                    lax.bitwise_and(w, jnp.int32(-65536)),
                                    jnp.float32)
                                a[bank + 2 * g2] = a[bank + 2 * g2] + lo
                                a[bank + 2 * g2 + 1] = (
                                    a[bank + 2 * g2 + 1] + hi)
                        return tuple(a)

                    accs = lax.fori_loop(0, _QTR // 4, acc_body, accs)
                    if h == 3:
                        for k in range(2 * _VPR):
                            stage_v[s, pl.ds(k * _LANES, _LANES)] = accs[k]
                    nxt_t = t0 + b + _NBUF
                    nxt_s = s + _NBUF // 4
                    @pl.when(nxt_t < ntasks)
                    def _issue(_b=b, _h=h, _s=nxt_s):
                        pltpu.async_copy(
                            table_hbm.at[idx_v.at[_s, _h]],
                            rows_v.at[_b], sems.at[_b])
                return carry2

            lax.fori_loop(0, ntasks // _NBUF, group_body, 0)
            pltpu.sync_copy(stage_v, out_hbm.at[pl.ds(cbase, _CH)])
            return carry

        lax.fori_loop(0, nchunk, chunk_body, 0)

    return pl.kernel(
        body,
        out_type=jax.ShapeDtypeStruct((s_pad, 2 * _D), jnp.float32),
        mesh=mesh,
        compiler_params=pltpu.CompilerParams(use_tc_tiling_on_sc=False),
        scratch_types=[
            pltpu.VMEM((_CH, 4, _QTR), jnp.int32),
            pltpu.VMEM((_NBUF, _QTR, _D // 2), jnp.int32),
            pltpu.VMEM((_CH, 2 * _D), jnp.float32),
            pltpu.SemaphoreType.DMA((_NBUF,)),
        ],
    )


_PREC = lax.Precision.DEFAULT


def _dense_call(bsz, blk, nb, few, n_blk_off):
    grid = (bsz // blk,)
    d = _D
    dm = 2 * d
    dh = 2 * dm
    fnb = float(nb)

    def body(qlp, qrp, qld, qrd, slp, srp, sld, srd,
             gwT, gb, w1T, b1, w2T, b2, lng, lnb, wihT, whhT, bih, bhh,
             out_ref):
        dot = functools.partial(jnp.dot, preferred_element_type=jnp.float32,
                                precision=_PREC)
        gwTv = gwT[...]
        gbv = gb[...]

        def nenc(p, deg):
            return jnp.tanh((dot(p, gwTv) + fnb * gbv) / deg)

        lngv = lng[...]
        lnbv = lnb[...]
        w1Tv = w1T[...]
        b1v = b1[...]
        w2Tv = w2T[...]
        b2v = b2[...]

        def senc(x):
            hh = jnp.maximum(dot(x, w1Tv) + b1v, 0.0)
            hh = dot(hh, w2Tv) + b2v
            z = hh + x
            mu = jnp.mean(z, axis=-1, keepdims=True)
            var = jnp.sum((z - mu) ** 2, axis=-1, keepdims=True) / (dm - 1)
            return (z - mu) / (jnp.sqrt(var) + 1e-3) * lngv + lnbv

        sl = nenc(slp[...], sld[...])
        sr = nenc(srp[...], srd[...])
        sgall = senc(jnp.concatenate([sl, sr], axis=1))
        smask = (lax.broadcasted_iota(jnp.int32, sgall.shape, 0)
                 < few).astype(jnp.float32)
        sg = jnp.sum(sgall * smask, axis=0, keepdims=True) / float(few)

        ql = nenc(qlp[...], qld[...])
        qr = nenc(qrp[...], qrd[...])
        qg = senc(jnp.concatenate([ql, qr], axis=1))

        whhTv = whhT[...]
        xp = dot(qg, wihT[...]) + bih[...] + bhh[...]
        sterm = dot(sg, whhTv[dm:dh])
        whhTl = whhTv[0:dm]
        h = None
        c = None
        for step in range(4):
            gates = xp if step == 0 else xp + dot(h, whhTl) + sterm
            ig = jax.nn.sigmoid(gates[:, 0:dh])
            fg = jax.nn.sigmoid(gates[:, dh:2 * dh])
            gg = jnp.tanh(gates[:, 2 * dh:3 * dh])
            og = jax.nn.sigmoid(gates[:, 3 * dh:4 * dh])
            c = ig * gg if step == 0 else fg * c + ig * gg
            h = qg + (og * jnp.tanh(c))[:, 0:dm]
        out_ref[...] = jnp.sum(h * sg, axis=1, keepdims=True)

    row_spec = lambda cols: pl.BlockSpec((blk, cols), lambda i: (i, 0))
    full = lambda shape: pl.BlockSpec(shape, lambda i: (0,) * len(shape))
    return pl.pallas_call(
        body,
        grid=grid,
        in_specs=[
            pl.BlockSpec((blk, dm), lambda i: (i, 0)),
            pl.BlockSpec((blk, dm), lambda i: (i + n_blk_off, 0)),
            row_spec(1), row_spec(1),
            full((8, dm)), full((8, dm)), full((8, 1)), full((8, 1)),
            full((dm, d)), full((1, d)),
            full((dm, dh)), full((1, dh)),
            full((dh, dm)), full((1, dm)),
            full((1, dm)), full((1, dm)),
            full((dm, 4 * dh)), full((dh, 4 * dh)),
            full((1, 4 * dh)), full((1, 4 * dh)),
        ],
        out_specs=row_spec(1),
        out_shape=jax.ShapeDtypeStruct((bsz, 1), jnp.float32),
    )


def kernel(query, support, query_left_connections, query_left_degrees,
           query_right_connections, query_right_degrees,
           support_left_connections, support_left_degrees,
           support_right_connections, support_right_degrees,
           symbol_emb, gcn_w_W, gcn_w_b, se_w1, se_b1, se_w2, se_b2,
           ln_g, ln_b, qe_wih, qe_whh, qe_bih, qe_bhh):
    bsz = query_left_connections.shape[0]
    few = support_left_connections.shape[0]
    nb = query_left_connections.shape[1]

    n_zpad = 512
    table = jnp.concatenate(
        [symbol_emb, jnp.zeros((n_zpad, symbol_emb.shape[1]),
                               symbol_emb.dtype)], axis=0)
    zbase = symbol_emb.shape[0]
    rt = table.shape[0]
    tb = table.astype(jnp.bfloat16).reshape(rt, _D // 2, 2)
    table_p = lax.bitcast_convert_type(tb, jnp.int32)

    ids = jnp.concatenate([
        query_left_connections.reshape(bsz, 2 * nb),
        query_right_connections.reshape(bsz, 2 * nb),
        support_left_connections.reshape(few, 2 * nb),
        support_right_connections.reshape(few, 2 * nb),
    ], axis=0).astype(jnp.int32)
    s_raw = ids.shape[0]
    unit = _NW * _CH
    s_pad = -(-s_raw // unit) * unit
    dummy = jnp.zeros((s_pad - s_raw, 2 * nb), jnp.int32)
    dummy = zbase + (
        lax.broadcasted_iota(jnp.int32, dummy.shape, 0) * (2 * nb)
        + lax.broadcasted_iota(jnp.int32, dummy.shape, 1)) % n_zpad
    ids = jnp.concatenate([ids, dummy], axis=0).reshape(s_pad, 4, _QTR)

    pooled = _sc_pool_call(s_pad, rt)(ids, table_p)

    off = 2 * bsz
    pad_s = jnp.zeros((8 - few, 2 * _D), jnp.float32)
    slp = jnp.concatenate([pooled[off:off + few], pad_s], 0)
    srp = jnp.concatenate([pooled[off + few:off + 2 * few], pad_s], 0)
    pad_d = jnp.ones((8 - few, 1), jnp.float32)
    sld = jnp.concatenate([support_left_degrees.reshape(few, 1), pad_d], 0)
    srd = jnp.concatenate([support_right_degrees.reshape(few, 1), pad_d], 0)

    gwT = gcn_w_W.T
    gw_perm = jnp.concatenate([gwT[_PERM], gwT[_D + _PERM]], axis=0)

    blk = 512
    scores = _dense_call(bsz, blk, nb, few, bsz // blk)(
        pooled, pooled,
        query_left_degrees.reshape(bsz, 1),
        query_right_degrees.reshape(bsz, 1),
        slp, srp, sld, srd,
        gw_perm, gcn_w_b.reshape(1, _D),
        se_w1.T, se_b1.reshape(1, -1),
        se_w2.T, se_b2.reshape(1, -1),
        ln_g.reshape(1, -1), ln_b.reshape(1, -1),
        qe_wih.T, qe_whh.T,
        qe_bih.reshape(1, -1), qe_bhh.reshape(1, -1),
    )
    return scores.reshape(bsz)
